# baseline (speedup 1.0000x reference)
import functools

import jax
import jax.numpy as jnp
from jax import lax
from jax.experimental import pallas as pl
from jax.experimental.pallas import tpu as pltpu
from jax.experimental.pallas import tpu_sc as plsc

N_CTX = 8192
CH = 768
ED = 512
QROWS = 224
TOPK9 = 9
LANES = 16
SCALE = 14 ** 0.5

TN = 1024

NWORK = 32
GB = 8192
GPW = GB // NWORK
GCH = 64


def _qk_body(q_ref, wk_ref, qk_ref):
    qk_ref[...] = lax.dot_general(
        q_ref[...], wk_ref[...], (((1,), (1,)), ((), ())),
        preferred_element_type=jnp.float32)


def _attn_body(x_ref, qk_ref, attn_ref, xsum_ref):
    t = pl.program_id(1)
    xt = x_ref[0]
    at = lax.dot_general(qk_ref[...], xt, (((1,), (1,)), ((), ())),
                         preferred_element_type=jnp.float32)
    attn_ref[...] = at * SCALE

    @pl.when(t == 0)
    def _():
        xsum_ref[...] = jnp.zeros_like(xsum_ref)

    xsum_ref[0] += jnp.broadcast_to(
        jnp.sum(xt, axis=0, keepdims=True), (8, xt.shape[1]))


RPW = 32
NSB = 16
SBW = N_CTX // NSB


def _sc_topk(attn2):
    mesh = plsc.VectorSubcoreMesh(core_axis_name="c", subcore_axis_name="s")

    @functools.partial(
        pl.kernel,
        mesh=mesh,
        out_type=jax.ShapeDtypeStruct((GB,), jnp.int32),
        scratch_types=[
            pltpu.VMEM((2 * N_CTX,), jnp.float32),
            pltpu.VMEM((NSB * 16,), jnp.float32),
            pltpu.VMEM((NSB * 16,), jnp.int32),
            pltpu.VMEM((32,), jnp.float32),
            pltpu.VMEM((32,), jnp.int32),
            pltpu.VMEM((TOPK9 * RPW + 16,), jnp.int32),
            pltpu.SemaphoreType.DMA,
        ],
    )
    def k(attn_hbm, gidx_hbm, row_v, sb_v, sbi_v, pad_v, pad_i, idx_v, sem):
        wid = lax.axis_index("s") * 2 + lax.axis_index("c")
        lane = lax.iota(jnp.int32, 16)
        NEG = jnp.float32(-jnp.inf)
        BIG = jnp.int32(2 ** 30)
        zf = jnp.zeros((16,), jnp.float32)
        zi = jnp.zeros((16,), jnp.int32)

        @pl.when(wid < 28)
        def _():
            b = wid // 7
            c0 = (wid % 7) * RPW
            base_row = b * QROWS + c0

            def fold_sb(l, cur):
                base = l * SBW
                acc = row_v[pl.ds(cur + base, 16)]
                iacc = lane + base
                for kk in range(1, SBW // 16):
                    x = row_v[pl.ds(cur + base + kk * 16, 16)]
                    better = x > acc
                    acc = jnp.where(better, x, acc)
                    iacc = jnp.where(better, lane + (base + kk * 16), iacc)
                sb_v[pl.ds(l * 16, 16)] = acc
                sbi_v[pl.ds(l * 16, 16)] = iacc

            pltpu.async_copy(attn_hbm.at[base_row],
                             row_v.at[pl.ds(0, N_CTX)], sem)

            def do_row(r, carry):
                cur = (r % 2) * N_CTX
                pltpu.make_async_copy(
                    attn_hbm.at[base_row + r],
                    row_v.at[pl.ds(cur, N_CTX)], sem).wait()

                @pl.when(r + 1 < RPW)
                def _():
                    pltpu.async_copy(
                        attn_hbm.at[base_row + r + 1],
                        row_v.at[pl.ds(((r + 1) % 2) * N_CTX, N_CTX)], sem)

                def build(l, c2):
                    fold_sb(l, cur)
                    return c2

                lax.fori_loop(0, NSB, build, 0)
                pad_v[pl.ds(16, 16)] = zf + NEG
                pad_i[pl.ds(16, 16)] = zi + BIG

                def extract(j, carry2):
                    g = sb_v[pl.ds(0, 16)]
                    gi = sbi_v[pl.ds(0, 16)]
                    for l in range(1, NSB):
                        x = sb_v[pl.ds(l * 16, 16)]
                        xi = sbi_v[pl.ds(l * 16, 16)]
                        better = x > g
                        g = jnp.where(better, x, g)
                        gi = jnp.where(better, xi, gi)
                    pad_v[pl.ds(0, 16)] = g
                    pad_i[pl.ds(0, 16)] = gi
                    for s in (8, 4, 2, 1):
                        a = pad_v[pl.ds(0, 16)]
                        bb = pad_v[pl.ds(s, 16)]
                        ia = pad_i[pl.ds(0, 16)]
                        ib = pad_i[pl.ds(s, 16)]
                        better = (bb > a) | ((bb == a) & (ib < ia))
                        pad_v[pl.ds(0, 16)] = jnp.where(better, bb, a)
                        pad_i[pl.ds(0, 16)] = jnp.where(better, ib, ia)
                    ix = pad_i[pl.ds(0, 16)][0]
                    off = cur + (ix // 16) * 16
                    sel = lane == (ix % 16)
                    vv = row_v[pl.ds(off, 16)]
                    row_v[pl.ds(off, 16)] = jnp.where(sel, NEG, vv)
                    fold_sb(ix // SBW, cur)
                    off2 = j * RPW + (r // 16) * 16
                    sel2 = lane == (r % 16)
                    prev = idx_v[pl.ds(off2, 16)]
                    idx_v[pl.ds(off2, 16)] = jnp.where(
                        sel2, zi + (ix + b * N_CTX), prev)
                    return carry2

                lax.fori_loop(0, TOPK9, extract, 0)
                return carry

            lax.fori_loop(0, RPW, do_row, 0)
            for j in range(TOPK9):
                pltpu.sync_copy(
                    idx_v.at[pl.ds(j * RPW, RPW)],
                    gidx_hbm.at[pl.ds(b * (TOPK9 * QROWS) + j * QROWS + c0,
                                      RPW)])

        @pl.when(wid == 28)
        def _():
            for t in range(8):
                idx_v[pl.ds(t * 16, 16)] = jnp.zeros((16,), jnp.int32)
            pltpu.sync_copy(idx_v.at[pl.ds(0, GB - TOPK9 * QROWS * 4)],
                            gidx_hbm.at[pl.ds(TOPK9 * QROWS * 4,
                                              GB - TOPK9 * QROWS * 4)])

    return k(attn2)


def _sc_gather(table, idx):
    mesh = plsc.VectorSubcoreMesh(core_axis_name="c", subcore_axis_name="s")

    @functools.partial(
        pl.kernel,
        mesh=mesh,
        out_type=jax.ShapeDtypeStruct((GB, CH), jnp.float32),
        scratch_types=[
            pltpu.VMEM((2 * GCH,), jnp.int32),
            pltpu.VMEM((2 * GCH, CH), jnp.float32),
            pltpu.SemaphoreType.DMA,
            pltpu.SemaphoreType.DMA,
        ],
    )
    def k(table_hbm, idx_hbm, out_hbm, idx_v, rows_v, gsem, ssem):
        wid = lax.axis_index("s") * 2 + lax.axis_index("c")
        base = wid * GPW
        nch = GPW // GCH

        def idxs(i):
            return idx_v.at[pl.ds((i % 2) * GCH, GCH)]

        def rows(i):
            return rows_v.at[pl.ds((i % 2) * GCH, GCH)]

        pltpu.sync_copy(idx_hbm.at[pl.ds(base, GCH)], idxs(0))
        pltpu.async_copy(table_hbm.at[idxs(0)], rows(0), gsem)
        for i in range(nch):
            off = base + i * GCH
            if i + 1 < nch:
                if i >= 1:
                    pltpu.make_async_copy(
                        rows(i - 1),
                        out_hbm.at[pl.ds(off - GCH, GCH)], ssem).wait()
                pltpu.sync_copy(
                    idx_hbm.at[pl.ds(off + GCH, GCH)], idxs(i + 1))
                pltpu.async_copy(table_hbm.at[idxs(i + 1)], rows(i + 1), gsem)
            pltpu.make_async_copy(
                table_hbm.at[idxs(i)], rows(i), gsem).wait()
            pltpu.async_copy(rows(i), out_hbm.at[pl.ds(off, GCH)], ssem)
        pltpu.make_async_copy(
            rows(nch - 2),
            out_hbm.at[pl.ds(base + (nch - 2) * GCH, GCH)], ssem).wait()
        pltpu.make_async_copy(
            rows(nch - 1),
            out_hbm.at[pl.ds(base + (nch - 1) * GCH, GCH)], ssem).wait()

    return k(table, idx)


def _combine_body(xg_ref, xsum_ref, wv_ref, wk_ref, q_ref, wp_ref, bp_ref,
                  out_ref):
    lane = lax.broadcasted_iota(jnp.int32, (QROWS, LANES), 1)
    valid = lane < TOPK9

    ktop = jnp.dot(xg_ref[...], wk_ref[...],
                   preferred_element_type=jnp.float32)
    qv = q_ref[...]
    rowi = lax.broadcasted_iota(jnp.int32, (QROWS, QROWS), 0)
    coli = lax.broadcasted_iota(jnp.int32, (QROWS, QROWS), 1)
    tv = jnp.zeros((QROWS, LANES), jnp.float32)
    for j in range(TOPK9):
        kj = ktop[j * QROWS:(j + 1) * QROWS, :]
        sq = lax.dot_general(qv, kj, (((1,), (1,)), ((), ())),
                             preferred_element_type=jnp.float32)
        sj = jnp.sum(jnp.where(rowi == coli, sq, 0.0),
                     axis=1, keepdims=True) * SCALE
        tv = jnp.where(lane == j, sj, tv)

    nrm = jnp.sqrt(jnp.sum(tv * tv, axis=1, keepdims=True))
    s = tv / nrm * N_CTX
    s = jnp.where(valid, s, -jnp.inf)
    m = jnp.maximum(jnp.max(s, axis=1, keepdims=True), 0.0)
    e = jnp.where(valid, jnp.exp(s - m), 0.0)
    ebg = jnp.exp(-m)
    z = jnp.sum(e, axis=1, keepdims=True) + (N_CTX - TOPK9) * ebg
    w = e / z

    vtop = jnp.dot(xg_ref[...], wv_ref[...],
                   preferred_element_type=jnp.float32)

    acc = jnp.zeros((QROWS, ED), jnp.float32)
    st = jnp.zeros((QROWS, ED), jnp.float32)
    for j in range(TOPK9):
        vj = vtop[j * QROWS:(j + 1) * QROWS, :]
        wj = jnp.sum(jnp.where(lane == j, w, 0.0), axis=1, keepdims=True)
        acc = acc + wj * vj
        st = st + vj

    vsum = jnp.dot(xsum_ref[0, 0:1, :], wv_ref[...],
                   preferred_element_type=jnp.float32)
    out = acc + (ebg / z) * (vsum - st)
    res = (jnp.dot(out, wp_ref[...], preferred_element_type=jnp.float32)
           + bp_ref[...])
    out_ref[0] = res[:196, :]


def kernel(x, q, Wk, Wv, Wp, bp):
    B, N, C = x.shape
    cn = q.shape[0]
    qp = jnp.zeros((QROWS, ED), q.dtype).at[:cn].set(q)

    qk = pl.pallas_call(
        _qk_body,
        in_specs=[
            pl.BlockSpec((QROWS, ED), lambda: (0, 0)),
            pl.BlockSpec((C, ED), lambda: (0, 0)),
        ],
        out_specs=pl.BlockSpec((QROWS, C), lambda: (0, 0)),
        out_shape=jax.ShapeDtypeStruct((QROWS, C), jnp.float32),
    )(qp, Wk)

    attn, xsum = pl.pallas_call(
        _attn_body,
        grid=(B, N // TN),
        in_specs=[
            pl.BlockSpec((1, TN, C), lambda b, t: (b, t, 0)),
            pl.BlockSpec((QROWS, C), lambda b, t: (0, 0)),
        ],
        out_specs=[
            pl.BlockSpec((QROWS, TN), lambda b, t: (b, t)),
            pl.BlockSpec((1, 8, C), lambda b, t: (b, 0, 0)),
        ],
        out_shape=[
            jax.ShapeDtypeStruct((B * QROWS, N), jnp.float32),
            jax.ShapeDtypeStruct((B, 8, C), jnp.float32),
        ],
        compiler_params=pltpu.CompilerParams(
            dimension_semantics=("parallel", "arbitrary")),
    )(x, qk)

    gidx = _sc_topk(attn)

    xg = _sc_gather(x.reshape(B * N, C), gidx)

    out = pl.pallas_call(
        _combine_body,
        grid=(B,),
        in_specs=[
            pl.BlockSpec((TOPK9 * QROWS, C), lambda b: (b, 0)),
            pl.BlockSpec((1, 8, C), lambda b: (b, 0, 0)),
            pl.BlockSpec((C, ED), lambda b: (0, 0)),
            pl.BlockSpec((C, ED), lambda b: (0, 0)),
            pl.BlockSpec((QROWS, ED), lambda b: (0, 0)),
            pl.BlockSpec((ED, C), lambda b: (0, 0)),
            pl.BlockSpec((1, C), lambda b: (0, 0)),
        ],
        out_specs=pl.BlockSpec((1, cn, C), lambda b: (b, 0, 0)),
        out_shape=jax.ShapeDtypeStruct((B, cn, C), jnp.float32),
    )(xg, xsum, Wv, Wk, qp, Wp, bp.reshape(1, C))

    return out


# speedup vs baseline: 1.0474x; 1.0474x over previous
import functools

import jax
import jax.numpy as jnp
from jax import lax
from jax.experimental import pallas as pl
from jax.experimental.pallas import tpu as pltpu
from jax.experimental.pallas import tpu_sc as plsc

N_CTX = 8192
CH = 768
ED = 512
QROWS = 224
TOPK9 = 9
LANES = 16
SCALE = 14 ** 0.5

TN = 2048

NWORK = 32
GB = 8192
GPW = GB // NWORK
GCH = 64


def _qk_body(q_ref, wk_ref, qk_ref):
    qk_ref[...] = lax.dot_general(
        q_ref[...], wk_ref[...], (((1,), (1,)), ((), ())),
        preferred_element_type=jnp.float32)


def _attn_body(x_ref, qk_ref, attn_ref, xsum_ref):
    t = pl.program_id(1)
    xt = x_ref[0]
    at = lax.dot_general(qk_ref[...], xt, (((1,), (1,)), ((), ())),
                         preferred_element_type=jnp.float32)
    attn_ref[...] = at * SCALE

    @pl.when(t == 0)
    def _():
        xsum_ref[...] = jnp.zeros_like(xsum_ref)

    xsum_ref[0] += jnp.broadcast_to(
        jnp.sum(xt, axis=0, keepdims=True), (8, xt.shape[1]))


RPW = 32
NSB = 16
SBW = N_CTX // NSB


def _sc_topk(attn2):
    mesh = plsc.VectorSubcoreMesh(core_axis_name="c", subcore_axis_name="s")

    @functools.partial(
        pl.kernel,
        mesh=mesh,
        out_type=jax.ShapeDtypeStruct((GB,), jnp.int32),
        scratch_types=[
            pltpu.VMEM((2 * N_CTX,), jnp.float32),
            pltpu.VMEM((NSB * 16,), jnp.float32),
            pltpu.VMEM((NSB * 16,), jnp.int32),
            pltpu.VMEM((32,), jnp.float32),
            pltpu.VMEM((32,), jnp.int32),
            pltpu.VMEM((TOPK9 * RPW + 16,), jnp.int32),
            pltpu.SemaphoreType.DMA,
        ],
    )
    def k(attn_hbm, gidx_hbm, row_v, sb_v, sbi_v, pad_v, pad_i, idx_v, sem):
        wid = lax.axis_index("s") * 2 + lax.axis_index("c")
        lane = lax.iota(jnp.int32, 16)
        NEG = jnp.float32(-jnp.inf)
        BIG = jnp.int32(2 ** 30)
        zf = jnp.zeros((16,), jnp.float32)
        zi = jnp.zeros((16,), jnp.int32)

        @pl.when(wid < 28)
        def _():
            b = wid // 7
            c0 = (wid % 7) * RPW
            base_row = b * QROWS + c0

            def fold_sb(l, cur):
                base = l * SBW
                acc = row_v[pl.ds(cur + base, 16)]
                iacc = lane + base
                for kk in range(1, SBW // 16):
                    x = row_v[pl.ds(cur + base + kk * 16, 16)]
                    better = x > acc
                    acc = jnp.where(better, x, acc)
                    iacc = jnp.where(better, lane + (base + kk * 16), iacc)
                sb_v[pl.ds(l * 16, 16)] = acc
                sbi_v[pl.ds(l * 16, 16)] = iacc

            pltpu.async_copy(attn_hbm.at[base_row],
                             row_v.at[pl.ds(0, N_CTX)], sem)

            def do_row(r, carry):
                cur = (r % 2) * N_CTX
                pltpu.make_async_copy(
                    attn_hbm.at[base_row + r],
                    row_v.at[pl.ds(cur, N_CTX)], sem).wait()

                @pl.when(r + 1 < RPW)
                def _():
                    pltpu.async_copy(
                        attn_hbm.at[base_row + r + 1],
                        row_v.at[pl.ds(((r + 1) % 2) * N_CTX, N_CTX)], sem)

                def build(l, c2):
                    fold_sb(l, cur)
                    return c2

                lax.fori_loop(0, NSB, build, 0)
                pad_v[pl.ds(16, 16)] = zf + NEG
                pad_i[pl.ds(16, 16)] = zi + BIG

                def extract(j, carry2):
                    g = sb_v[pl.ds(0, 16)]
                    gi = sbi_v[pl.ds(0, 16)]
                    for l in range(1, NSB):
                        x = sb_v[pl.ds(l * 16, 16)]
                        xi = sbi_v[pl.ds(l * 16, 16)]
                        better = x > g
                        g = jnp.where(better, x, g)
                        gi = jnp.where(better, xi, gi)
                    pad_v[pl.ds(0, 16)] = g
                    pad_i[pl.ds(0, 16)] = gi
                    for s in (8, 4, 2, 1):
                        a = pad_v[pl.ds(0, 16)]
                        bb = pad_v[pl.ds(s, 16)]
                        ia = pad_i[pl.ds(0, 16)]
                        ib = pad_i[pl.ds(s, 16)]
                        better = (bb > a) | ((bb == a) & (ib < ia))
                        pad_v[pl.ds(0, 16)] = jnp.where(better, bb, a)
                        pad_i[pl.ds(0, 16)] = jnp.where(better, ib, ia)
                    ix = pad_i[pl.ds(0, 16)][0]
                    off = cur + (ix // 16) * 16
                    sel = lane == (ix % 16)
                    vv = row_v[pl.ds(off, 16)]
                    row_v[pl.ds(off, 16)] = jnp.where(sel, NEG, vv)
                    fold_sb(ix // SBW, cur)
                    off2 = j * RPW + (r // 16) * 16
                    sel2 = lane == (r % 16)
                    prev = idx_v[pl.ds(off2, 16)]
                    idx_v[pl.ds(off2, 16)] = jnp.where(
                        sel2, zi + (ix + b * N_CTX), prev)
                    return carry2

                lax.fori_loop(0, TOPK9, extract, 0)
                return carry

            lax.fori_loop(0, RPW, do_row, 0)
            for j in range(TOPK9):
                pltpu.sync_copy(
                    idx_v.at[pl.ds(j * RPW, RPW)],
                    gidx_hbm.at[pl.ds(b * (TOPK9 * QROWS) + j * QROWS + c0,
                                      RPW)])

        @pl.when(wid == 28)
        def _():
            for t in range(8):
                idx_v[pl.ds(t * 16, 16)] = jnp.zeros((16,), jnp.int32)
            pltpu.sync_copy(idx_v.at[pl.ds(0, GB - TOPK9 * QROWS * 4)],
                            gidx_hbm.at[pl.ds(TOPK9 * QROWS * 4,
                                              GB - TOPK9 * QROWS * 4)])

    return k(attn2)


def _sc_gather(table, idx):
    mesh = plsc.VectorSubcoreMesh(core_axis_name="c", subcore_axis_name="s")

    @functools.partial(
        pl.kernel,
        mesh=mesh,
        out_type=jax.ShapeDtypeStruct((GB, CH), jnp.float32),
        scratch_types=[
            pltpu.VMEM((2 * GCH,), jnp.int32),
            pltpu.VMEM((2 * GCH, CH), jnp.float32),
            pltpu.SemaphoreType.DMA,
            pltpu.SemaphoreType.DMA,
        ],
    )
    def k(table_hbm, idx_hbm, out_hbm, idx_v, rows_v, gsem, ssem):
        wid = lax.axis_index("s") * 2 + lax.axis_index("c")
        base = wid * GPW
        nch = GPW // GCH

        def idxs(i):
            return idx_v.at[pl.ds((i % 2) * GCH, GCH)]

        def rows(i):
            return rows_v.at[pl.ds((i % 2) * GCH, GCH)]

        pltpu.sync_copy(idx_hbm.at[pl.ds(base, GCH)], idxs(0))
        pltpu.async_copy(table_hbm.at[idxs(0)], rows(0), gsem)
        for i in range(nch):
            off = base + i * GCH
            if i + 1 < nch:
                if i >= 1:
                    pltpu.make_async_copy(
                        rows(i - 1),
                        out_hbm.at[pl.ds(off - GCH, GCH)], ssem).wait()
                pltpu.sync_copy(
                    idx_hbm.at[pl.ds(off + GCH, GCH)], idxs(i + 1))
                pltpu.async_copy(table_hbm.at[idxs(i + 1)], rows(i + 1), gsem)
            pltpu.make_async_copy(
                table_hbm.at[idxs(i)], rows(i), gsem).wait()
            pltpu.async_copy(rows(i), out_hbm.at[pl.ds(off, GCH)], ssem)
        pltpu.make_async_copy(
            rows(nch - 2),
            out_hbm.at[pl.ds(base + (nch - 2) * GCH, GCH)], ssem).wait()
        pltpu.make_async_copy(
            rows(nch - 1),
            out_hbm.at[pl.ds(base + (nch - 1) * GCH, GCH)], ssem).wait()

    return k(table, idx)


def _combine_body(xg_ref, xsum_ref, wv_ref, wk_ref, q_ref, wp_ref, bp_ref,
                  out_ref):
    lane = lax.broadcasted_iota(jnp.int32, (QROWS, LANES), 1)
    valid = lane < TOPK9

    ktop = jnp.dot(xg_ref[...], wk_ref[...],
                   preferred_element_type=jnp.float32)
    qv = q_ref[...]
    rowi = lax.broadcasted_iota(jnp.int32, (QROWS, QROWS), 0)
    coli = lax.broadcasted_iota(jnp.int32, (QROWS, QROWS), 1)
    tv = jnp.zeros((QROWS, LANES), jnp.float32)
    for j in range(TOPK9):
        kj = ktop[j * QROWS:(j + 1) * QROWS, :]
        sq = lax.dot_general(qv, kj, (((1,), (1,)), ((), ())),
                             preferred_element_type=jnp.float32)
        sj = jnp.sum(jnp.where(rowi == coli, sq, 0.0),
                     axis=1, keepdims=True) * SCALE
        tv = jnp.where(lane == j, sj, tv)

    nrm = jnp.sqrt(jnp.sum(tv * tv, axis=1, keepdims=True))
    s = tv / nrm * N_CTX
    s = jnp.where(valid, s, -jnp.inf)
    m = jnp.maximum(jnp.max(s, axis=1, keepdims=True), 0.0)
    e = jnp.where(valid, jnp.exp(s - m), 0.0)
    ebg = jnp.exp(-m)
    z = jnp.sum(e, axis=1, keepdims=True) + (N_CTX - TOPK9) * ebg
    w = e / z

    vtop = jnp.dot(xg_ref[...], wv_ref[...],
                   preferred_element_type=jnp.float32)

    acc = jnp.zeros((QROWS, ED), jnp.float32)
    st = jnp.zeros((QROWS, ED), jnp.float32)
    for j in range(TOPK9):
        vj = vtop[j * QROWS:(j + 1) * QROWS, :]
        wj = jnp.sum(jnp.where(lane == j, w, 0.0), axis=1, keepdims=True)
        acc = acc + wj * vj
        st = st + vj

    vsum = jnp.dot(xsum_ref[0, 0:1, :], wv_ref[...],
                   preferred_element_type=jnp.float32)
    out = acc + (ebg / z) * (vsum - st)
    res = (jnp.dot(out, wp_ref[...], preferred_element_type=jnp.float32)
           + bp_ref[...])
    out_ref[0] = res[:196, :]


def kernel(x, q, Wk, Wv, Wp, bp):
    B, N, C = x.shape
    cn = q.shape[0]
    qp = jnp.zeros((QROWS, ED), q.dtype).at[:cn].set(q)

    qk = pl.pallas_call(
        _qk_body,
        in_specs=[
            pl.BlockSpec((QROWS, ED), lambda: (0, 0)),
            pl.BlockSpec((C, ED), lambda: (0, 0)),
        ],
        out_specs=pl.BlockSpec((QROWS, C), lambda: (0, 0)),
        out_shape=jax.ShapeDtypeStruct((QROWS, C), jnp.float32),
    )(qp, Wk)

    attn, xsum = pl.pallas_call(
        _attn_body,
        grid=(B, N // TN),
        in_specs=[
            pl.BlockSpec((1, TN, C), lambda b, t: (b, t, 0)),
            pl.BlockSpec((QROWS, C), lambda b, t: (0, 0)),
        ],
        out_specs=[
            pl.BlockSpec((QROWS, TN), lambda b, t: (b, t)),
            pl.BlockSpec((1, 8, C), lambda b, t: (b, 0, 0)),
        ],
        out_shape=[
            jax.ShapeDtypeStruct((B * QROWS, N), jnp.float32),
            jax.ShapeDtypeStruct((B, 8, C), jnp.float32),
        ],
        compiler_params=pltpu.CompilerParams(
            dimension_semantics=("parallel", "arbitrary")),
    )(x, qk)

    gidx = _sc_topk(attn)

    xg = _sc_gather(x.reshape(B * N, C), gidx)

    out = pl.pallas_call(
        _combine_body,
        grid=(B,),
        in_specs=[
            pl.BlockSpec((TOPK9 * QROWS, C), lambda b: (b, 0)),
            pl.BlockSpec((1, 8, C), lambda b: (b, 0, 0)),
            pl.BlockSpec((C, ED), lambda b: (0, 0)),
            pl.BlockSpec((C, ED), lambda b: (0, 0)),
            pl.BlockSpec((QROWS, ED), lambda b: (0, 0)),
            pl.BlockSpec((ED, C), lambda b: (0, 0)),
            pl.BlockSpec((1, C), lambda b: (0, 0)),
        ],
        out_specs=pl.BlockSpec((1, cn, C), lambda b: (b, 0, 0)),
        out_shape=jax.ShapeDtypeStruct((B, cn, C), jnp.float32),
    )(xg, xsum, Wv, Wk, qp, Wp, bp.reshape(1, C))

    return out
